# baseline (speedup 1.0000x reference)
import jax
import jax.numpy as jnp
from jax.experimental import pallas as pl
from jax.experimental.pallas import tpu as pltpu

_BATCH = 4096
_DIM = 128
_NBLK = 4
_ROWS = _BATCH // _NBLK


def _copy_body(x_ref, o_ref):
    o_ref[...] = x_ref[...]


def kernel(initial_space, finite_space, time_embedding):
    return pl.pallas_call(
        _copy_body,
        grid=(_NBLK,),
        in_specs=[pl.BlockSpec((_ROWS, _DIM), lambda i: (i, 0))],
        out_specs=pl.BlockSpec((_ROWS, _DIM), lambda i: (i, 0)),
        out_shape=jax.ShapeDtypeStruct((_BATCH, _DIM), jnp.float32),
        compiler_params=pltpu.CompilerParams(
            dimension_semantics=("arbitrary",),
        ),
    )(initial_space)


# speedup vs baseline: 1.0548x; 1.0548x over previous
import jax
import jax.numpy as jnp
from jax.experimental import pallas as pl
from jax.experimental.pallas import tpu as pltpu

_BATCH = 4096
_DIM = 128
_NCHUNK = 4
_ROWS = _BATCH // _NCHUNK


def _copy_body(x_ref, o_ref, buf, in_sems, out_sems):
    def in_copy(i):
        return pltpu.make_async_copy(
            x_ref.at[pl.ds(i * _ROWS, _ROWS)], buf.at[i], in_sems.at[i])

    def out_copy(i):
        return pltpu.make_async_copy(
            buf.at[i], o_ref.at[pl.ds(i * _ROWS, _ROWS)], out_sems.at[i])

    in_copy(0).start()
    for i in range(_NCHUNK):
        if i + 1 < _NCHUNK:
            in_copy(i + 1).start()
        in_copy(i).wait()
        out_copy(i).start()
    for i in range(_NCHUNK):
        out_copy(i).wait()


def kernel(initial_space, finite_space, time_embedding):
    return pl.pallas_call(
        _copy_body,
        in_specs=[pl.BlockSpec(memory_space=pl.ANY)],
        out_specs=pl.BlockSpec(memory_space=pl.ANY),
        out_shape=jax.ShapeDtypeStruct((_BATCH, _DIM), jnp.float32),
        scratch_shapes=[
            pltpu.VMEM((_NCHUNK, _ROWS, _DIM), jnp.float32),
            pltpu.SemaphoreType.DMA((_NCHUNK,)),
            pltpu.SemaphoreType.DMA((_NCHUNK,)),
        ],
    )(initial_space)


# speedup vs baseline: 1.4754x; 1.3988x over previous
import jax
import jax.numpy as jnp
from jax.experimental import pallas as pl
from jax.experimental.pallas import tpu as pltpu

_BATCH = 4096
_DIM = 128
_NCHUNK = 2
_ROWS = _BATCH // _NCHUNK


def _copy_body(x_ref, o_ref, buf, in_sems, out_sems):
    def in_copy(i):
        return pltpu.make_async_copy(
            x_ref.at[pl.ds(i * _ROWS, _ROWS)], buf.at[i], in_sems.at[i])

    def out_copy(i):
        return pltpu.make_async_copy(
            buf.at[i], o_ref.at[pl.ds(i * _ROWS, _ROWS)], out_sems.at[i])

    in_copy(0).start()
    for i in range(_NCHUNK):
        if i + 1 < _NCHUNK:
            in_copy(i + 1).start()
        in_copy(i).wait()
        out_copy(i).start()
    for i in range(_NCHUNK):
        out_copy(i).wait()


def kernel(initial_space, finite_space, time_embedding):
    return pl.pallas_call(
        _copy_body,
        in_specs=[pl.BlockSpec(memory_space=pl.ANY)],
        out_specs=pl.BlockSpec(memory_space=pl.ANY),
        out_shape=jax.ShapeDtypeStruct((_BATCH, _DIM), jnp.float32),
        scratch_shapes=[
            pltpu.VMEM((_NCHUNK, _ROWS, _DIM), jnp.float32),
            pltpu.SemaphoreType.DMA((_NCHUNK,)),
            pltpu.SemaphoreType.DMA((_NCHUNK,)),
        ],
    )(initial_space)
